# baseline (speedup 1.0000x reference)
import functools

import jax
import jax.numpy as jnp
from jax import lax
from jax.experimental import pallas as pl
from jax.experimental.pallas import tpu as pltpu
from jax.experimental.pallas import tpu_sc as plsc

H = 768
T = 32768
NM = 4000
NN = 2000
F = 256
MW = 10
BT = 1024
NBLK = T // BT
BSEG = 4096 // BT


RNG = T // 32


@functools.partial(
    pl.kernel,
    out_type=jax.ShapeDtypeStruct((T,), jnp.int32),
    mesh=plsc.VectorSubcoreMesh(core_axis_name="c", subcore_axis_name="s"),
    scratch_types=[
        pltpu.VMEM((NM,), jnp.int32),
        pltpu.VMEM((RNG,), jnp.int32),
    ],
    compiler_params=pltpu.CompilerParams(needs_layout_passes=False),
)
def _build_sel(midx_hbm, zeros_hbm, sel_hbm, idx_v, sel_v):
    cid = lax.axis_index("c")
    sid = lax.axis_index("s")
    wid = sid * 2 + cid
    lo = wid * RNG

    pltpu.sync_copy(zeros_hbm, sel_v)
    pltpu.sync_copy(midx_hbm, idx_v)
    lane = lax.iota(jnp.int32, 16)

    def body(i, carry):
        rank = lane + i * 16
        val = rank % MW + 1
        idx16 = idx_v[pl.ds(i * 16, 16)]
        inrange = (idx16 >= lo) & (idx16 < lo + RNG)
        plsc.store_scatter(sel_v, [idx16 - lo], val, mask=inrange)
        return carry

    lax.fori_loop(0, NM // 16, body, 0)
    pltpu.sync_copy(sel_v, sel_hbm.at[pl.ds(lo, RNG)])


def _fuse_body(sel_ref, x_hbm, embt_ref, wq_ref, bq_ref, xout_ref, y_ref,
               xbuf, sems):
    i = pl.program_id(0)

    def start(j, slot):
        pltpu.make_async_copy(
            x_hbm.at[:, j // BSEG, pl.ds((j % BSEG) * BT, BT)],
            xbuf.at[slot], sems.at[slot]).start()

    @pl.when(i == 0)
    def _():
        start(0, 0)
        start(1, 1)
        start(2, 2)

    @pl.when(i + 3 < NBLK)
    def _():
        start(i + 3, (i + 3) % 4)

    slot = i % 4
    pltpu.make_async_copy(
        x_hbm.at[:, i // BSEG, pl.ds((i % BSEG) * BT, BT)],
        xbuf.at[slot], sems.at[slot]).wait()

    sel = sel_ref[0]
    iot = lax.broadcasted_iota(jnp.int32, (16, BT), 0)
    onehot = (iot == sel).astype(jnp.float32)
    embcols = jnp.dot(embt_ref[...], onehot,
                      preferred_element_type=jnp.float32)
    out = jnp.where(sel == 0, xbuf[slot], embcols)
    xout_ref[...] = out
    y_ref[...] = lax.dot_general(
        out, wq_ref[...], (((0,), (0,)), ((), ())),
        preferred_element_type=jnp.float32) + bq_ref[...]


def _fuse(sel3, xview, embt, wq, bq2):
    return pl.pallas_call(
        _fuse_body,
        grid=(NBLK,),
        in_specs=[
            pl.BlockSpec((1, 1, BT), lambda i: (i, 0, 0)),
            pl.BlockSpec(memory_space=pl.ANY),
            pl.BlockSpec((H, 16), lambda i: (0, 0)),
            pl.BlockSpec((H, F), lambda i: (0, 0)),
            pl.BlockSpec((1, F), lambda i: (0, 0)),
        ],
        out_specs=[
            pl.BlockSpec((H, BT), lambda i: (0, i)),
            pl.BlockSpec((BT, F), lambda i: (i, 0)),
        ],
        out_shape=[
            jax.ShapeDtypeStruct((H, T), jnp.float32),
            jax.ShapeDtypeStruct((T, F), jnp.float32),
        ],
        scratch_shapes=[
            pltpu.VMEM((4, H, BT), jnp.float32),
            pltpu.SemaphoreType.DMA((4,)),
        ],
        compiler_params=pltpu.CompilerParams(
            fuse_transposed_lhs_in_matmul=True),
    )(sel3, xview, embt, wq, bq2)


CH = 200


@functools.partial(
    pl.kernel,
    out_type=[
        jax.ShapeDtypeStruct((NM, F), jnp.float32),
        jax.ShapeDtypeStruct((NN, F), jnp.float32),
    ],
    mesh=plsc.VectorSubcoreMesh(core_axis_name="c", subcore_axis_name="s"),
    scratch_types=[
        pltpu.VMEM((CH,), jnp.int32),
        pltpu.VMEM((CH, F), jnp.float32),
        pltpu.SemaphoreType.DMA,
    ],
    compiler_params=pltpu.CompilerParams(needs_layout_passes=False),
)
def _gather_rows(y_hbm, midx_hbm, nidx_hbm, m_hbm, n_hbm, idx_v, rows_v, sem):
    cid = lax.axis_index("c")
    sid = lax.axis_index("s")
    wid = sid * 2 + cid

    def job(idx_hbm, out_hbm, base):
        pltpu.sync_copy(idx_hbm.at[pl.ds(base, CH)], idx_v)
        pltpu.async_copy(y_hbm.at[idx_v.at[pl.ds(0, 128)]],
                         rows_v.at[pl.ds(0, 128)], sem).wait()
        pltpu.async_copy(y_hbm.at[idx_v.at[pl.ds(128, 72)]],
                         rows_v.at[pl.ds(128, 72)], sem).wait()
        pltpu.sync_copy(rows_v, out_hbm.at[pl.ds(base, CH)])

    @pl.when(wid < 20)
    def _():
        job(midx_hbm, m_hbm, wid * CH)

    @pl.when((wid >= 20) & (wid < 30))
    def _():
        job(nidx_hbm, n_hbm, (wid - 20) * CH)


def kernel(inputs, mask_emb_weight, Wq, bq, mask_idx, neg_idx):
    xview = inputs.reshape(H, 8, 4096)

    sel = _build_sel(mask_idx, jnp.zeros((RNG,), jnp.int32))
    sel3 = sel.reshape(NBLK, 1, BT)

    embt = jnp.pad(mask_emb_weight, ((1, 16 - MW - 1), (0, 0))).T
    xout, y = _fuse(sel3, xview, embt, Wq, bq.reshape(1, F))

    m_out, n_out = _gather_rows(y, mask_idx, neg_idx)

    return xout.reshape(1, H, T), m_out[None], n_out[None]


# speedup vs baseline: 1.0167x; 1.0167x over previous
import functools

import jax
import jax.numpy as jnp
from jax import lax
from jax.experimental import pallas as pl
from jax.experimental.pallas import tpu as pltpu
from jax.experimental.pallas import tpu_sc as plsc

H = 768
T = 32768
NM = 4000
NN = 2000
F = 256
MW = 10
BT = 2048
NBLK = T // BT
BSEG = 4096 // BT


RNG = T // 32


@functools.partial(
    pl.kernel,
    out_type=jax.ShapeDtypeStruct((T,), jnp.int32),
    mesh=plsc.VectorSubcoreMesh(core_axis_name="c", subcore_axis_name="s"),
    scratch_types=[
        pltpu.VMEM((NM,), jnp.int32),
        pltpu.VMEM((RNG,), jnp.int32),
    ],
    compiler_params=pltpu.CompilerParams(needs_layout_passes=False),
)
def _build_sel(midx_hbm, zeros_hbm, sel_hbm, idx_v, sel_v):
    cid = lax.axis_index("c")
    sid = lax.axis_index("s")
    wid = sid * 2 + cid
    lo = wid * RNG

    pltpu.sync_copy(zeros_hbm, sel_v)
    pltpu.sync_copy(midx_hbm, idx_v)
    lane = lax.iota(jnp.int32, 16)

    def body(i, carry):
        rank = lane + i * 16
        val = rank % MW + 1
        idx16 = idx_v[pl.ds(i * 16, 16)]
        inrange = (idx16 >= lo) & (idx16 < lo + RNG)
        plsc.store_scatter(sel_v, [idx16 - lo], val, mask=inrange)
        return carry

    lax.fori_loop(0, NM // 16, body, 0)
    pltpu.sync_copy(sel_v, sel_hbm.at[pl.ds(lo, RNG)])


def _fuse_body(sel_ref, x_hbm, embt_ref, wq_ref, bq_ref, xout_ref, y_ref,
               m_ref, xbuf, sems):
    i = pl.program_id(0)

    def start(j, slot):
        pltpu.make_async_copy(
            x_hbm.at[:, j // BSEG, pl.ds((j % BSEG) * BT, BT)],
            xbuf.at[slot], sems.at[slot]).start()

    @pl.when(i == 0)
    def _():
        start(0, 0)
        start(1, 1)

    @pl.when(i + 2 < NBLK)
    def _():
        start(i + 2, (i + 2) % 3)

    slot = i % 3
    pltpu.make_async_copy(
        x_hbm.at[:, i // BSEG, pl.ds((i % BSEG) * BT, BT)],
        xbuf.at[slot], sems.at[slot]).wait()

    sel = sel_ref[0]
    iot = lax.broadcasted_iota(jnp.int32, (16, BT), 0)
    onehot = (iot == sel).astype(jnp.float32)
    embcols = jnp.dot(embt_ref[...], onehot,
                      preferred_element_type=jnp.float32)
    out = jnp.where(sel == 0, xbuf[slot], embcols)
    xout_ref[...] = out
    y_ref[...] = lax.dot_general(
        out, wq_ref[...], (((0,), (0,)), ((), ())),
        preferred_element_type=jnp.float32) + bq_ref[...]

    @pl.when(i == NBLK - 1)
    def _():
        m16 = lax.dot_general(
            embt_ref[...], wq_ref[...], (((0,), (0,)), ((), ())),
            preferred_element_type=jnp.float32)
        m10 = m16[1:MW + 1] + bq_ref[...]
        m_ref[...] = jnp.broadcast_to(
            m10[None], (NM // MW, MW, F)).reshape(NM, F)


def _fuse(sel3, xview, embt, wq, bq2):
    return pl.pallas_call(
        _fuse_body,
        grid=(NBLK,),
        in_specs=[
            pl.BlockSpec((1, 1, BT), lambda i: (i, 0, 0)),
            pl.BlockSpec(memory_space=pl.ANY),
            pl.BlockSpec((H, 16), lambda i: (0, 0)),
            pl.BlockSpec((H, F), lambda i: (0, 0)),
            pl.BlockSpec((1, F), lambda i: (0, 0)),
        ],
        out_specs=[
            pl.BlockSpec((H, BT), lambda i: (0, i)),
            pl.BlockSpec((BT, F), lambda i: (i, 0)),
            pl.BlockSpec((NM, F), lambda i: (0, 0)),
        ],
        out_shape=[
            jax.ShapeDtypeStruct((H, T), jnp.float32),
            jax.ShapeDtypeStruct((T, F), jnp.float32),
            jax.ShapeDtypeStruct((NM, F), jnp.float32),
        ],
        scratch_shapes=[
            pltpu.VMEM((3, H, BT), jnp.float32),
            pltpu.SemaphoreType.DMA((3,)),
        ],
        compiler_params=pltpu.CompilerParams(
            fuse_transposed_lhs_in_matmul=True),
    )(sel3, xview, embt, wq, bq2)


CH = 80


@functools.partial(
    pl.kernel,
    out_type=jax.ShapeDtypeStruct((NN, F), jnp.float32),
    mesh=plsc.VectorSubcoreMesh(core_axis_name="c", subcore_axis_name="s"),
    scratch_types=[
        pltpu.VMEM((CH,), jnp.int32),
        pltpu.VMEM((CH, F), jnp.float32),
        pltpu.SemaphoreType.DMA,
    ],
    compiler_params=pltpu.CompilerParams(needs_layout_passes=False),
)
def _gather_rows(y_hbm, nidx_hbm, n_hbm, idx_v, rows_v, sem):
    cid = lax.axis_index("c")
    sid = lax.axis_index("s")
    wid = sid * 2 + cid

    @pl.when(wid < NN // CH)
    def _():
        base = wid * CH
        pltpu.sync_copy(nidx_hbm.at[pl.ds(base, CH)], idx_v)
        pltpu.async_copy(y_hbm.at[idx_v], rows_v, sem).wait()
        pltpu.sync_copy(rows_v, n_hbm.at[pl.ds(base, CH)])


def kernel(inputs, mask_emb_weight, Wq, bq, mask_idx, neg_idx):
    xview = inputs.reshape(H, 8, 4096)

    sel = _build_sel(mask_idx, jnp.zeros((RNG,), jnp.int32))
    sel3 = sel.reshape(NBLK, 1, BT)

    embt = jnp.pad(mask_emb_weight, ((1, 16 - MW - 1), (0, 0))).T
    xout, y, m_out = _fuse(sel3, xview, embt, Wq, bq.reshape(1, F))

    n_out = _gather_rows(y, neg_idx)

    return xout.reshape(1, H, T), m_out[None], n_out[None]
